# baseline (speedup 1.0000x reference)
import functools

import jax
import jax.numpy as jnp
from jax import lax
from jax.experimental import pallas as pl
from jax.experimental.pallas import tpu as pltpu
from jax.experimental.pallas import tpu_sc as plsc

N_NODES = 10000
N_EDGES = 320000
D = 128

NC = 2
NS = 16
NW = NC * NS
EPW = N_EDGES // NW
B = 125
NB = EPW // B
CB = 16
NCH = NB // CB
N_PAD = 10112
RPT = N_PAD // NS
CW = 16


def _sc_aggregate_body(src_hbm, dst_hbm, x_hbm, zagg_hbm, zcnt_hbm,
                       ones_hbm, aggp_out, cntp_out,
                       src_v, dst_v, r0, r1, r2, r3, r4, ones_v,
                       agg_sh, cnt_sh, s0, s1, s2, s3, s4):
    c = lax.axis_index("c")
    s = lax.axis_index("s")
    wid = c * NS + s
    rows = (r0, r1, r2, r3, r4)
    sems = (s0, s1, s2, s3, s4)

    pltpu.sync_copy(src_hbm.at[wid], src_v)
    pltpu.sync_copy(dst_hbm.at[wid], dst_v)
    pltpu.sync_copy(ones_hbm, ones_v)

    pltpu.sync_copy(zagg_hbm, agg_sh.at[pl.ds(s * RPT, RPT)])
    pltpu.sync_copy(zcnt_hbm, cnt_sh.at[pl.ds(s * RPT, RPT)])
    plsc.subcore_barrier()

    def start_gather(j, u):
        pltpu.make_async_copy(x_hbm.at[src_v.at[j]], rows[u], sems[u]).start()

    start_gather(0, 0)
    start_gather(1, 1)
    start_gather(2, 2)
    start_gather(3, 3)

    @pl.loop(0, NB, step=5)
    def _(j):
        for u in range(5):
            jj = j + u
            pltpu.make_async_copy(x_hbm.at[src_v.at[jj]], rows[u],
                                  sems[u]).wait()
            pltpu.sync_copy(rows[u], agg_sh.at[dst_v.at[jj]], add=True)
            pltpu.sync_copy(ones_v, cnt_sh.at[dst_v.at[jj]], add=True)

            @pl.when(jj + 4 < NB)
            def _():
                start_gather(jj + 4, (u + 4) % 5)

    plsc.subcore_barrier()

    pltpu.sync_copy(agg_sh.at[pl.ds(s * RPT, RPT)],
                    aggp_out.at[c, pl.ds(s * RPT, RPT)])
    pltpu.sync_copy(cnt_sh.at[pl.ds(s * RPT, RPT)],
                    cntp_out.at[c, pl.ds(s * RPT, RPT)])


_sc_aggregate = pl.kernel(
    _sc_aggregate_body,
    out_type=(
        jax.ShapeDtypeStruct((NC, N_PAD, D), jnp.bfloat16),
        jax.ShapeDtypeStruct((NC, N_PAD, CW), jnp.float32),
    ),
    mesh=plsc.VectorSubcoreMesh(core_axis_name="c", subcore_axis_name="s"),
    compiler_params=pltpu.CompilerParams(use_tc_tiling_on_sc=False),
    scratch_types=[
        pltpu.VMEM((NB, B), jnp.int32),
        pltpu.VMEM((NB, B), jnp.int32),
        pltpu.VMEM((B, D), jnp.bfloat16),
        pltpu.VMEM((B, D), jnp.bfloat16),
        pltpu.VMEM((B, D), jnp.bfloat16),
        pltpu.VMEM((B, D), jnp.bfloat16),
        pltpu.VMEM((B, D), jnp.bfloat16),
        pltpu.VMEM((B, CW), jnp.float32),
        pltpu.VMEM_SHARED((N_PAD, D), jnp.bfloat16),
        pltpu.VMEM_SHARED((N_PAD, CW), jnp.float32),
        pltpu.SemaphoreType.DMA,
        pltpu.SemaphoreType.DMA,
        pltpu.SemaphoreType.DMA,
        pltpu.SemaphoreType.DMA,
        pltpu.SemaphoreType.DMA,
    ],
)


def _tc_combine_body(x_ref, aggp_ref, cntp_ref, wl_ref, wr_ref, b_ref,
                     out_ref):
    agg = (aggp_ref[0].astype(jnp.float32) +
           aggp_ref[1].astype(jnp.float32))
    cnt = cntp_ref[0, :, 0:1] + cntp_ref[1, :, 0:1]
    mean = agg / jnp.maximum(cnt, 1.0)
    dn = (((1,), (1,)), ((), ()))
    acc = lax.dot_general(mean, wl_ref[...], dn,
                          preferred_element_type=jnp.float32)
    acc += lax.dot_general(x_ref[...], wr_ref[...], dn,
                           preferred_element_type=jnp.float32)
    out_ref[...] = jnp.maximum(acc + b_ref[...], 0.0)


_TC_ROWS = 1000


def _tc_combine(x, aggp, cntp, wl_t, wr_t, b2d):
    grid = (N_NODES // _TC_ROWS,)
    return pl.pallas_call(
        _tc_combine_body,
        grid=grid,
        in_specs=[
            pl.BlockSpec((_TC_ROWS, D), lambda i: (i, 0)),
            pl.BlockSpec((NC, _TC_ROWS, D), lambda i: (0, i, 0)),
            pl.BlockSpec((NC, _TC_ROWS, CW), lambda i: (0, i, 0)),
            pl.BlockSpec((D, D), lambda i: (0, 0)),
            pl.BlockSpec((D, D), lambda i: (0, 0)),
            pl.BlockSpec((1, D), lambda i: (0, 0)),
        ],
        out_specs=pl.BlockSpec((_TC_ROWS, D), lambda i: (i, 0)),
        out_shape=jax.ShapeDtypeStruct((N_NODES, D), jnp.float32),
    )(x, aggp, cntp, wl_t, wr_t, b2d)


def kernel(x, edge_index, W_l, b_l, W_r):
    src = edge_index[0].reshape(NW, NB, B)
    dst = edge_index[1].reshape(NW, NB, B)
    zagg = jnp.zeros((RPT, D), jnp.bfloat16)
    zcnt = jnp.zeros((RPT, CW), jnp.float32)
    ones_col = jnp.zeros((B, CW), jnp.float32).at[:, 0].set(1.0)
    xb = x.astype(jnp.bfloat16)
    aggp, cntp = _sc_aggregate(src, dst, xb, zagg, zcnt, ones_col)
    return _tc_combine(x, aggp, cntp, W_l, W_r, b_l.reshape(1, D))


# speedup vs baseline: 1.0912x; 1.0912x over previous
import functools

import jax
import jax.numpy as jnp
from jax import lax
from jax.experimental import pallas as pl
from jax.experimental.pallas import tpu as pltpu
from jax.experimental.pallas import tpu_sc as plsc

N_NODES = 10000
N_EDGES = 320000
D = 128

NC = 2
NS = 16
NW = NC * NS
EPW = N_EDGES // NW
B = 125
NB = EPW // B
CB = 16
NCH = NB // CB
N_PAD = 10112
RPT = N_PAD // NS
CW = 16


def _sc_aggregate_body(edges_hbm, x_hbm, zagg_hbm, zcnt_hbm,
                       ones_hbm, aggp_out, cntp_out,
                       src_v, dst_v, r0, r1, r2, r3, r4, ones_v,
                       agg_sh, cnt_sh, s0, s1, s2, s3, s4):
    c = lax.axis_index("c")
    s = lax.axis_index("s")
    wid = c * NS + s
    rows = (r0, r1, r2, r3, r4)
    sems = (s0, s1, s2, s3, s4)

    pltpu.sync_copy(edges_hbm.at[0, wid], src_v)
    pltpu.sync_copy(edges_hbm.at[1, wid], dst_v)
    pltpu.sync_copy(ones_hbm, ones_v)

    pltpu.sync_copy(zagg_hbm, agg_sh.at[pl.ds(s * RPT, RPT)])
    pltpu.sync_copy(zcnt_hbm, cnt_sh.at[pl.ds(s * RPT, RPT)])
    plsc.subcore_barrier()

    def start_gather(j, u):
        pltpu.make_async_copy(x_hbm.at[src_v.at[j]], rows[u], sems[u]).start()

    start_gather(0, 0)
    start_gather(1, 1)
    start_gather(2, 2)
    start_gather(3, 3)

    @pl.loop(0, NB, step=5)
    def _(j):
        for u in range(5):
            jj = j + u
            pltpu.make_async_copy(x_hbm.at[src_v.at[jj]], rows[u],
                                  sems[u]).wait()
            pltpu.sync_copy(rows[u], agg_sh.at[dst_v.at[jj]], add=True)
            pltpu.sync_copy(ones_v, cnt_sh.at[dst_v.at[jj]], add=True)

            @pl.when(jj + 4 < NB)
            def _():
                start_gather(jj + 4, (u + 4) % 5)

    plsc.subcore_barrier()

    pltpu.sync_copy(agg_sh.at[pl.ds(s * RPT, RPT)],
                    aggp_out.at[c, pl.ds(s * RPT, RPT)])
    pltpu.sync_copy(cnt_sh.at[pl.ds(s * RPT, RPT)],
                    cntp_out.at[c, pl.ds(s * RPT, RPT)])


_sc_aggregate = pl.kernel(
    _sc_aggregate_body,
    out_type=(
        jax.ShapeDtypeStruct((NC, N_PAD, D), jnp.bfloat16),
        jax.ShapeDtypeStruct((NC, N_PAD, CW), jnp.float32),
    ),
    mesh=plsc.VectorSubcoreMesh(core_axis_name="c", subcore_axis_name="s"),
    compiler_params=pltpu.CompilerParams(use_tc_tiling_on_sc=False),
    scratch_types=[
        pltpu.VMEM((NB, B), jnp.int32),
        pltpu.VMEM((NB, B), jnp.int32),
        pltpu.VMEM((B, D), jnp.bfloat16),
        pltpu.VMEM((B, D), jnp.bfloat16),
        pltpu.VMEM((B, D), jnp.bfloat16),
        pltpu.VMEM((B, D), jnp.bfloat16),
        pltpu.VMEM((B, D), jnp.bfloat16),
        pltpu.VMEM((B, CW), jnp.float32),
        pltpu.VMEM_SHARED((N_PAD, D), jnp.bfloat16),
        pltpu.VMEM_SHARED((N_PAD, CW), jnp.float32),
        pltpu.SemaphoreType.DMA,
        pltpu.SemaphoreType.DMA,
        pltpu.SemaphoreType.DMA,
        pltpu.SemaphoreType.DMA,
        pltpu.SemaphoreType.DMA,
    ],
)


def _tc_combine_body(x_ref, aggp_ref, cntp_ref, wl_ref, wr_ref, b_ref,
                     out_ref):
    agg = (aggp_ref[0].astype(jnp.float32) +
           aggp_ref[1].astype(jnp.float32))
    cnt = cntp_ref[0, :, 0:1] + cntp_ref[1, :, 0:1]
    mean = agg / jnp.maximum(cnt, 1.0)
    dn = (((1,), (1,)), ((), ()))
    acc = lax.dot_general(mean, wl_ref[...], dn,
                          preferred_element_type=jnp.float32)
    acc += lax.dot_general(x_ref[...], wr_ref[...], dn,
                           preferred_element_type=jnp.float32)
    out_ref[...] = jnp.maximum(acc + b_ref[...], 0.0)


_TC_ROWS = 1000


def _tc_combine(x, aggp, cntp, wl_t, wr_t, b2d):
    grid = (N_NODES // _TC_ROWS,)
    return pl.pallas_call(
        _tc_combine_body,
        grid=grid,
        in_specs=[
            pl.BlockSpec((_TC_ROWS, D), lambda i: (i, 0)),
            pl.BlockSpec((NC, _TC_ROWS, D), lambda i: (0, i, 0)),
            pl.BlockSpec((NC, _TC_ROWS, CW), lambda i: (0, i, 0)),
            pl.BlockSpec((D, D), lambda i: (0, 0)),
            pl.BlockSpec((D, D), lambda i: (0, 0)),
            pl.BlockSpec((1, D), lambda i: (0, 0)),
        ],
        out_specs=pl.BlockSpec((_TC_ROWS, D), lambda i: (i, 0)),
        out_shape=jax.ShapeDtypeStruct((N_NODES, D), jnp.float32),
    )(x, aggp, cntp, wl_t, wr_t, b2d)


def kernel(x, edge_index, W_l, b_l, W_r):
    edges = edge_index.reshape(2, NW, NB, B)
    zagg = jnp.zeros((RPT, D), jnp.bfloat16)
    zcnt = jnp.zeros((RPT, CW), jnp.float32)
    ones_col = jnp.zeros((B, CW), jnp.float32).at[:, 0].set(1.0)
    xb = x.astype(jnp.bfloat16)
    aggp, cntp = _sc_aggregate(edges, xb, zagg, zcnt, ones_col)
    return _tc_combine(x, aggp, cntp, W_l, W_r, b_l.reshape(1, D))
